# baseline (speedup 1.0000x reference)
import functools

import jax
import jax.numpy as jnp
from jax import lax
from jax.experimental import pallas as pl
from jax.experimental.pallas import tpu as pltpu
from jax.experimental.pallas import tpu_sc as plsc

_Q = 1024
_D = 16
_K = 100000
_BK = 4096
_KP = 102400
_NB = _KP // _BK
_RS = 64
_NS = _Q // _RS
_NC = _BK // 128

_NW = 32
_QW = _Q // _NW
_NCD = _QW * _NC


def _pass1_kernel(q2_ref, kt_ref, out_ref, qsqb_ref, minval, minblk, minlane):
    kb = pl.program_id(0)

    @pl.when(kb == 0)
    def _():
        q = q2_ref[...] * 0.5
        qsq = jnp.sum(q * q, axis=1, keepdims=True)
        qsqb_ref[...] = jnp.broadcast_to(qsq, (_Q, 128))
        minval[...] = jnp.full((_Q, 1), 3.0e38, jnp.float32)
        minblk[...] = jnp.zeros((_Q, 1), jnp.int32)
        minlane[...] = jnp.zeros((_Q, 1), jnp.int32)

    kt = kt_ref[...]
    ksq = jnp.sum(kt * kt, axis=0, keepdims=True)
    dot = jnp.dot(q2_ref[...], kt, preferred_element_type=jnp.float32)

    for s in range(_NS):
        rs = slice(s * _RS, (s + 1) * _RS)
        qb = qsqb_ref[rs, :]
        m = (qb + ksq[:, 0:128]) - dot[rs, 0:128]
        for c in range(1, _NC):
            m = jnp.minimum(m, (qb + ksq[:, c * 128:(c + 1) * 128])
                            - dot[rs, c * 128:(c + 1) * 128])
        tmin = jnp.min(m, axis=1, keepdims=True)
        lane = jax.lax.broadcasted_iota(jnp.int32, (_RS, 128), 1)
        tlane = jnp.min(jnp.where(m == tmin, lane, jnp.int32(2**30)),
                        axis=1, keepdims=True)
        mv = minval[rs, :]
        better = tmin < mv
        minblk[rs, :] = jnp.where(better, kb, minblk[rs, :])
        minlane[rs, :] = jnp.where(better, tlane, minlane[rs, :])
        minval[rs, :] = jnp.where(better, tmin, mv)

    @pl.when(kb == _NB - 1)
    def _():
        out_ref[...] = minblk[...] * _BK + minlane[...]


def _pass1(q2, kt):
    return pl.pallas_call(
        _pass1_kernel,
        grid=(_NB,),
        in_specs=[
            pl.BlockSpec((_Q, _D), lambda kb: (0, 0)),
            pl.BlockSpec((_D, _BK), lambda kb: (0, kb)),
        ],
        out_specs=pl.BlockSpec((_Q, 1), lambda kb: (0, 0)),
        out_shape=jax.ShapeDtypeStruct((_Q, 1), jnp.int32),
        scratch_shapes=[
            pltpu.VMEM((_Q, 128), jnp.float32),
            pltpu.VMEM((_Q, 1), jnp.float32),
            pltpu.VMEM((_Q, 1), jnp.int32),
            pltpu.VMEM((_Q, 1), jnp.int32),
        ],
    )(q2, kt)


def _pass2_body(*refs):
    cols_hbm = refs[:_D]
    q2cols_hbm = refs[_D]
    base_hbm = refs[_D + 1]
    out_hbm = refs[_D + 2]
    base_v, idx_all, kcols, q2w, out_v, sem = refs[_D + 3:]

    wid = lax.axis_index("s") * 2 + lax.axis_index("c")
    qbase = wid * _QW
    pltpu.sync_copy(base_hbm.at[pl.ds(qbase, _QW)], base_v)
    for d in range(_D):
        pltpu.sync_copy(q2cols_hbm.at[pl.ds(d * _Q + qbase, _QW)],
                        q2w.at[pl.ds(d * _QW, _QW)])

    for g in range(_QW // 16):
        base16 = base_v[pl.ds(g * 16, 16)]
        for c in range(_NC):
            idx_all[pl.ds(c * _QW + g * 16, 16)] = base16 + c * 128
    copies = [
        pltpu.async_copy(cols_hbm[d].at[idx_all],
                         kcols.at[pl.ds(d * _NCD, _NCD)], sem)
        for d in range(_D)
    ]
    for cp in copies:
        cp.wait()

    for g in range(_QW // 16):
        qsq = jnp.zeros((16,), jnp.float32)
        qcols = []
        for d in range(_D):
            qd = q2w[pl.ds(d * _QW + g * 16, 16)]
            qcols.append(qd)
            qh = qd * 0.5
            qsq = qsq + qh * qh
        base16 = base_v[pl.ds(g * 16, 16)]
        best = jnp.full((16,), 3.0e38, jnp.float32)
        bestidx = jnp.zeros((16,), jnp.int32)
        for c in range(_NC):
            acc = jnp.zeros((16,), jnp.float32)
            ksq = jnp.zeros((16,), jnp.float32)
            for d in range(_D):
                kd = kcols[pl.ds(d * _NCD + c * _QW + g * 16, 16)]
                acc = acc + qcols[d] * kd
                ksq = ksq + kd * kd
            d2 = (qsq + ksq) - acc
            lt = d2 < best
            best = jnp.where(lt, d2, best)
            bestidx = jnp.where(lt, base16 + c * 128, bestidx)
        out_v[pl.ds(g * 16, 16)] = bestidx
    pltpu.sync_copy(out_v, out_hbm.at[pl.ds(qbase, _QW)])


def _pass2(kt, q2t_flat, base):
    mesh = plsc.VectorSubcoreMesh(core_axis_name="c", subcore_axis_name="s")
    fn = functools.partial(
        pl.kernel, mesh=mesh,
        out_type=jax.ShapeDtypeStruct((_Q,), jnp.int32),
        scratch_types=[
            pltpu.VMEM((_QW,), jnp.int32),
            pltpu.VMEM((_NCD,), jnp.int32),
            pltpu.VMEM((_D * _NCD,), jnp.float32),
            pltpu.VMEM((_D * _QW,), jnp.float32),
            pltpu.VMEM((_QW,), jnp.int32),
            pltpu.SemaphoreType.DMA,
        ],
    )(_pass2_body)
    return fn(*[kt[d] for d in range(_D)], q2t_flat, base)


def kernel(coords, keys):
    kt = jnp.pad(keys, ((0, _KP - _K), (0, 0)), constant_values=1000.0).T
    q2 = coords * 2.0
    q2t_flat = q2.T.reshape(-1)
    base = _pass1(q2, kt)[:, 0]
    return _pass2(kt, q2t_flat, base)


# speedup vs baseline: 1.4907x; 1.4907x over previous
import jax
import jax.numpy as jnp
from jax.experimental import pallas as pl
from jax.experimental.pallas import tpu as pltpu

_Q = 1024
_D = 16
_K = 100000
_BK = 4096
_KP = 102400
_NB = _KP // _BK
_RS = 64
_NS = _Q // _RS
_NC = _BK // 128


def _knn_kernel(q2_ref, kt_ref, out_ref, qsqb_ref, dot_ref, minval, minblk, minloc):
    kb = pl.program_id(0)

    @pl.when(kb == 0)
    def _():
        q = q2_ref[...] * 0.5
        qsq = jnp.sum(q * q, axis=1, keepdims=True)
        qsqb_ref[...] = jnp.broadcast_to(qsq, (_Q, 128))
        minval[...] = jnp.full((_Q, 1), 3.0e38, jnp.float32)
        minblk[...] = jnp.zeros((_Q, 1), jnp.int32)
        minloc[...] = jnp.zeros((_Q, 1), jnp.int32)

    kt = kt_ref[...]
    ksq = jnp.sum(kt * kt, axis=0, keepdims=True)
    dot_ref[...] = jnp.dot(q2_ref[...], kt, preferred_element_type=jnp.float32)

    for s in range(_NS):
        rs = slice(s * _RS, (s + 1) * _RS)
        qb = qsqb_ref[rs, :]
        m = (qb + ksq[:, 0:128]) - dot_ref[rs, 0:128]
        cid = jnp.zeros((_RS, 128), jnp.int32)
        for c in range(1, _NC):
            d2c = (qb + ksq[:, c * 128:(c + 1) * 128]) - dot_ref[rs, c * 128:(c + 1) * 128]
            lt = d2c < m
            m = jnp.where(lt, d2c, m)
            cid = jnp.where(lt, c, cid)
        tmin = jnp.min(m, axis=1, keepdims=True)
        lane = jax.lax.broadcasted_iota(jnp.int32, (_RS, 128), 1)
        j = cid * 128 + lane
        tloc = jnp.min(jnp.where(m == tmin, j, jnp.int32(2**30)),
                       axis=1, keepdims=True)
        mv = minval[rs, :]
        better = tmin < mv
        minblk[rs, :] = jnp.where(better, kb, minblk[rs, :])
        minloc[rs, :] = jnp.where(better, tloc, minloc[rs, :])
        minval[rs, :] = jnp.where(better, tmin, mv)

    @pl.when(kb == _NB - 1)
    def _():
        out_ref[...] = minblk[...] * _BK + minloc[...]


def kernel(coords, keys):
    kt = jnp.pad(keys, ((0, _KP - _K), (0, 0)), constant_values=1000.0).T
    q2 = coords * 2.0
    out = pl.pallas_call(
        _knn_kernel,
        grid=(_NB,),
        in_specs=[
            pl.BlockSpec((_Q, _D), lambda kb: (0, 0)),
            pl.BlockSpec((_D, _BK), lambda kb: (0, kb)),
        ],
        out_specs=pl.BlockSpec((_Q, 1), lambda kb: (0, 0)),
        out_shape=jax.ShapeDtypeStruct((_Q, 1), jnp.int32),
        scratch_shapes=[
            pltpu.VMEM((_Q, 128), jnp.float32),
            pltpu.VMEM((_Q, _BK), jnp.float32),
            pltpu.VMEM((_Q, 1), jnp.float32),
            pltpu.VMEM((_Q, 1), jnp.int32),
            pltpu.VMEM((_Q, 1), jnp.int32),
        ],
    )(q2, kt)
    return out[:, 0]


# speedup vs baseline: 1.5261x; 1.0237x over previous
import jax
import jax.numpy as jnp
from jax.experimental import pallas as pl
from jax.experimental.pallas import tpu as pltpu

_Q = 1024
_D = 16
_K = 100000
_BK = 5120
_KP = 102400
_NB = _KP // _BK
_RS = 64
_NS = _Q // _RS
_NC = _BK // 128


def _knn_kernel(q2_ref, kt_ref, out_ref, qsqb_ref, dot_ref, minval, minblk, minloc):
    kb = pl.program_id(0)

    @pl.when(kb == 0)
    def _():
        q = q2_ref[...] * 0.5
        qsq = jnp.sum(q * q, axis=1, keepdims=True)
        qsqb_ref[...] = jnp.broadcast_to(qsq, (_Q, 128))
        minval[...] = jnp.full((_Q, 1), 3.0e38, jnp.float32)
        minblk[...] = jnp.zeros((_Q, 1), jnp.int32)
        minloc[...] = jnp.zeros((_Q, 1), jnp.int32)

    kt = kt_ref[...]
    ksq = jnp.sum(kt * kt, axis=0, keepdims=True)
    dot_ref[...] = jnp.dot(q2_ref[...], kt, preferred_element_type=jnp.float32)

    for s in range(_NS):
        rs = slice(s * _RS, (s + 1) * _RS)
        qb = qsqb_ref[rs, :]
        m = (qb + ksq[:, 0:128]) - dot_ref[rs, 0:128]
        cid = jnp.zeros((_RS, 128), jnp.int32)
        for c in range(1, _NC):
            d2c = (qb + ksq[:, c * 128:(c + 1) * 128]) - dot_ref[rs, c * 128:(c + 1) * 128]
            lt = d2c < m
            m = jnp.where(lt, d2c, m)
            cid = jnp.where(lt, c, cid)
        tmin = jnp.min(m, axis=1, keepdims=True)
        lane = jax.lax.broadcasted_iota(jnp.int32, (_RS, 128), 1)
        j = cid * 128 + lane
        tloc = jnp.min(jnp.where(m == tmin, j, jnp.int32(2**30)),
                       axis=1, keepdims=True)
        mv = minval[rs, :]
        better = tmin < mv
        minblk[rs, :] = jnp.where(better, kb, minblk[rs, :])
        minloc[rs, :] = jnp.where(better, tloc, minloc[rs, :])
        minval[rs, :] = jnp.where(better, tmin, mv)

    @pl.when(kb == _NB - 1)
    def _():
        out_ref[...] = minblk[...] * _BK + minloc[...]


def kernel(coords, keys):
    kt = jnp.pad(keys, ((0, _KP - _K), (0, 0)), constant_values=1000.0).T
    q2 = coords * 2.0
    out = pl.pallas_call(
        _knn_kernel,
        grid=(_NB,),
        in_specs=[
            pl.BlockSpec((_Q, _D), lambda kb: (0, 0)),
            pl.BlockSpec((_D, _BK), lambda kb: (0, kb)),
        ],
        out_specs=pl.BlockSpec((_Q, 1), lambda kb: (0, 0)),
        out_shape=jax.ShapeDtypeStruct((_Q, 1), jnp.int32),
        scratch_shapes=[
            pltpu.VMEM((_Q, 128), jnp.float32),
            pltpu.VMEM((_Q, _BK), jnp.float32),
            pltpu.VMEM((_Q, 1), jnp.float32),
            pltpu.VMEM((_Q, 1), jnp.int32),
            pltpu.VMEM((_Q, 1), jnp.int32),
        ],
    )(q2, kt)
    return out[:, 0]


# speedup vs baseline: 1.5390x; 1.0085x over previous
import jax
import jax.numpy as jnp
from jax.experimental import pallas as pl
from jax.experimental.pallas import tpu as pltpu

_Q = 1024
_D = 16
_K = 100000
_BK = 6400
_KP = 102400
_NB = _KP // _BK
_RS = 64
_NS = _Q // _RS
_NC = _BK // 128


def _knn_kernel(q2_ref, kt_ref, out_ref, qsqb_ref, dot_ref, minval, minblk, minloc):
    kb = pl.program_id(0)

    @pl.when(kb == 0)
    def _():
        q = q2_ref[...] * 0.5
        qsq = jnp.sum(q * q, axis=1, keepdims=True)
        qsqb_ref[...] = jnp.broadcast_to(qsq, (_Q, 128))
        minval[...] = jnp.full((_Q, 1), 3.0e38, jnp.float32)
        minblk[...] = jnp.zeros((_Q, 1), jnp.int32)
        minloc[...] = jnp.zeros((_Q, 1), jnp.int32)

    kt = kt_ref[...]
    ksq = jnp.sum(kt * kt, axis=0, keepdims=True)
    dot_ref[...] = jnp.dot(q2_ref[...], kt, preferred_element_type=jnp.float32)

    for s in range(_NS):
        rs = slice(s * _RS, (s + 1) * _RS)
        qb = qsqb_ref[rs, :]
        m = (qb + ksq[:, 0:128]) - dot_ref[rs, 0:128]
        cid = jnp.zeros((_RS, 128), jnp.int32)
        for c in range(1, _NC):
            d2c = (qb + ksq[:, c * 128:(c + 1) * 128]) - dot_ref[rs, c * 128:(c + 1) * 128]
            lt = d2c < m
            m = jnp.where(lt, d2c, m)
            cid = jnp.where(lt, c, cid)
        tmin = jnp.min(m, axis=1, keepdims=True)
        lane = jax.lax.broadcasted_iota(jnp.int32, (_RS, 128), 1)
        j = cid * 128 + lane
        tloc = jnp.min(jnp.where(m == tmin, j, jnp.int32(2**30)),
                       axis=1, keepdims=True)
        mv = minval[rs, :]
        better = tmin < mv
        minblk[rs, :] = jnp.where(better, kb, minblk[rs, :])
        minloc[rs, :] = jnp.where(better, tloc, minloc[rs, :])
        minval[rs, :] = jnp.where(better, tmin, mv)

    @pl.when(kb == _NB - 1)
    def _():
        out_ref[...] = minblk[...] * _BK + minloc[...]


def kernel(coords, keys):
    kt = jnp.pad(keys, ((0, _KP - _K), (0, 0)), constant_values=1000.0).T
    q2 = coords * 2.0
    out = pl.pallas_call(
        _knn_kernel,
        grid=(_NB,),
        in_specs=[
            pl.BlockSpec((_Q, _D), lambda kb: (0, 0)),
            pl.BlockSpec((_D, _BK), lambda kb: (0, kb)),
        ],
        out_specs=pl.BlockSpec((_Q, 1), lambda kb: (0, 0)),
        out_shape=jax.ShapeDtypeStruct((_Q, 1), jnp.int32),
        scratch_shapes=[
            pltpu.VMEM((_Q, 128), jnp.float32),
            pltpu.VMEM((_Q, _BK), jnp.float32),
            pltpu.VMEM((_Q, 1), jnp.float32),
            pltpu.VMEM((_Q, 1), jnp.int32),
            pltpu.VMEM((_Q, 1), jnp.int32),
        ],
    )(q2, kt)
    return out[:, 0]
